# baseline (speedup 1.0000x reference)
import jax
import jax.numpy as jnp
from jax import lax
from jax.experimental import pallas as pl
from jax.experimental.pallas import tpu as pltpu
from jax.sharding import PartitionSpec as P


_TILE = 8192
_UNROLL = 512
_CHUNK = 2048
_VMEM_LIMIT = 48 << 20


def _gather_kernel(ids_ref, table_ref, out_ref, ids_smem, sems):
    tile = ids_ref.shape[-1]
    n_chunk = tile // _CHUNK

    def _copy(c):
        return pltpu.make_async_copy(
            ids_ref.at[0, 0, pl.ds(c * _CHUNK, _CHUNK)],
            ids_smem.at[pl.ds(c * _CHUNK, _CHUNK)],
            sems.at[c],
        )

    _copy(0).start()
    for c in range(n_chunk):
        if c + 1 < n_chunk:
            _copy(c + 1).start()
        _copy(c).wait()

        def body(j, carry, c=c):
            base = c * _CHUNK + j * _UNROLL
            for mi in range(_UNROLL):
                idx2 = pl.multiple_of(ids_smem[base + mi], 2)
                out_ref[pl.ds(pl.multiple_of((base + mi) * 2, 2), 2), :] = (
                    table_ref[pl.ds(idx2, 2), :]
                )
            return carry

        lax.fori_loop(0, _CHUNK // _UNROLL, body, 0)


def _gather_tiles(ids3, table2):
    n_tiles, _, tile = ids3.shape
    V2, _ = table2.shape
    return pl.pallas_call(
        _gather_kernel,
        out_shape=jax.ShapeDtypeStruct((n_tiles * tile * 2, 128), jnp.float32),
        grid=(n_tiles,),
        in_specs=[
            pl.BlockSpec((1, 1, tile), lambda i: (i, 0, 0)),
            pl.BlockSpec((V2, 128), lambda i: (0, 0),
                         pipeline_mode=pl.Buffered(1)),
        ],
        out_specs=pl.BlockSpec((tile * 2, 128), lambda i: (i, 0)),
        scratch_shapes=[
            pltpu.SMEM((tile,), jnp.int32),
            pltpu.SemaphoreType.DMA((tile // _CHUNK,)),
        ],
        compiler_params=pltpu.CompilerParams(
            dimension_semantics=("arbitrary",),
            vmem_limit_bytes=_VMEM_LIMIT,
        ),
    )(ids3, table2)


@jax.jit
def _emoji_gather(emojis, table):
    B, S = emojis.shape
    V, D = table.shape
    n = B * S
    p = D // 128

    ids = emojis.reshape(-1).astype(jnp.int32) * p
    tile = _TILE if n >= _TILE else pl.cdiv(n, _CHUNK) * _CHUNK
    n_pad = pl.cdiv(n, tile) * tile
    n_tiles = n_pad // tile
    if n_pad != n:
        ids = jnp.pad(ids, (0, n_pad - n))
    ids3 = ids.reshape(n_tiles, 1, tile)
    table2 = table.astype(jnp.float32).reshape(V * p, 128)

    ndev = len(jax.devices())
    if ndev > 1 and n_pad == n and n_tiles % ndev == 0 and B % ndev == 0:
        mesh = jax.make_mesh(
            (ndev,), ("c",),
            axis_types=(jax.sharding.AxisType.Explicit,),
        )
        ids3 = jax.reshard(ids3, jax.NamedSharding(mesh, P("c", None, None)))
        table2 = jax.reshard(table2, jax.NamedSharding(mesh, P()))

        def _local(i3, t2):
            return _gather_tiles(i3, t2).reshape(B // ndev, S, D)

        return jax.shard_map(
            _local, mesh=mesh, in_specs=(P("c"), P()), out_specs=P("c"),
            check_vma=False,
        )(ids3, table2)

    out = _gather_tiles(ids3, table2)
    if n_pad != n:
        out = out[: n * p]
    return out.reshape(B, S, D)


def kernel(emojis, table):
    return _emoji_gather(emojis, table)


# speedup vs baseline: 1.8110x; 1.8110x over previous
import jax
import jax.numpy as jnp
from jax import lax
from jax.experimental import pallas as pl
from jax.experimental.pallas import tpu as pltpu
from jax.sharding import PartitionSpec as P


_TILE = 8192
_UNROLL = 512
_CHUNK = 2048
_VMEM_LIMIT = 48 << 20


def _gather_kernel(ids_ref, table_ref, out_ref, ids_smem, sems):
    tile = ids_ref.shape[-1]
    n_chunk = tile // _CHUNK

    def _copy(c):
        return pltpu.make_async_copy(
            ids_ref.at[0, 0, pl.ds(c * _CHUNK, _CHUNK)],
            ids_smem.at[pl.ds(c * _CHUNK, _CHUNK)],
            sems.at[c],
        )

    _copy(0).start()
    for c in range(n_chunk):
        if c + 1 < n_chunk:
            _copy(c + 1).start()
        _copy(c).wait()

        def body(j, carry, c=c):
            base = c * _CHUNK + j * _UNROLL
            for g in range(_UNROLL // 8):
                rows = []
                for k in range(8):
                    idx2 = pl.multiple_of(ids_smem[base + g * 8 + k], 2)
                    rows.append(table_ref[pl.ds(idx2, 2), :])
                lo = jnp.concatenate([r[0:1, :] for r in rows], axis=0)
                hi = jnp.concatenate([r[1:2, :] for r in rows], axis=0)
                out_ref[pl.ds(pl.multiple_of(base + g * 8, 8), 8), 0:128] = lo
                out_ref[pl.ds(pl.multiple_of(base + g * 8, 8), 8), 128:256] = hi
            return carry

        lax.fori_loop(0, _CHUNK // _UNROLL, body, 0)


def _gather_tiles(ids3, table2):
    n_tiles, _, tile = ids3.shape
    V2, _ = table2.shape
    return pl.pallas_call(
        _gather_kernel,
        out_shape=jax.ShapeDtypeStruct((n_tiles * tile, 256), jnp.float32),
        grid=(n_tiles,),
        in_specs=[
            pl.BlockSpec((1, 1, tile), lambda i: (i, 0, 0)),
            pl.BlockSpec((V2, 128), lambda i: (0, 0),
                         pipeline_mode=pl.Buffered(1)),
        ],
        out_specs=pl.BlockSpec((tile, 256), lambda i: (i, 0)),
        scratch_shapes=[
            pltpu.SMEM((tile,), jnp.int32),
            pltpu.SemaphoreType.DMA((tile // _CHUNK,)),
        ],
        compiler_params=pltpu.CompilerParams(
            dimension_semantics=("arbitrary",),
            vmem_limit_bytes=_VMEM_LIMIT,
        ),
    )(ids3, table2)


@jax.jit
def _emoji_gather(emojis, table):
    B, S = emojis.shape
    V, D = table.shape
    n = B * S
    p = D // 128

    ids = emojis.reshape(-1).astype(jnp.int32) * p
    tile = _TILE if n >= _TILE else pl.cdiv(n, _CHUNK) * _CHUNK
    n_pad = pl.cdiv(n, tile) * tile
    n_tiles = n_pad // tile
    if n_pad != n:
        ids = jnp.pad(ids, (0, n_pad - n))
    ids3 = ids.reshape(n_tiles, 1, tile)
    table2 = table.astype(jnp.float32).reshape(V * p, 128)

    ndev = len(jax.devices())
    if ndev > 1 and n_pad == n and n_tiles % ndev == 0 and B % ndev == 0:
        mesh = jax.make_mesh(
            (ndev,), ("c",),
            axis_types=(jax.sharding.AxisType.Explicit,),
        )
        ids3 = jax.reshard(ids3, jax.NamedSharding(mesh, P("c", None, None)))
        table2 = jax.reshard(table2, jax.NamedSharding(mesh, P()))

        def _local(i3, t2):
            return _gather_tiles(i3, t2).reshape(B // ndev, S, D)

        return jax.shard_map(
            _local, mesh=mesh, in_specs=(P("c"), P()), out_specs=P("c"),
            check_vma=False,
        )(ids3, table2)

    out = _gather_tiles(ids3, table2)
    if n_pad != n:
        out = out[:n]
    return out.reshape(B, S, D)


def kernel(emojis, table):
    return _emoji_gather(emojis, table)


# speedup vs baseline: 1.8819x; 1.0392x over previous
import jax
import jax.numpy as jnp
from jax import lax
from jax.experimental import pallas as pl
from jax.experimental.pallas import tpu as pltpu
from jax.sharding import PartitionSpec as P


_TILE = 16384
_UNROLL = 512
_CHUNK = 2048
_VMEM_LIMIT = 48 << 20


def _gather_kernel(ids_ref, table_ref, out_ref, ids_smem, sems):
    tile = ids_ref.shape[-1]
    n_chunk = tile // _CHUNK

    def _copy(c):
        return pltpu.make_async_copy(
            ids_ref.at[0, 0, pl.ds(c * _CHUNK, _CHUNK)],
            ids_smem.at[pl.ds(c * _CHUNK, _CHUNK)],
            sems.at[c],
        )

    _copy(0).start()
    for c in range(n_chunk):
        if c + 1 < n_chunk:
            _copy(c + 1).start()
        _copy(c).wait()

        def body(j, carry, c=c):
            base = c * _CHUNK + j * _UNROLL
            for g in range(_UNROLL // 8):
                rows = []
                for k in range(8):
                    idx2 = pl.multiple_of(ids_smem[base + g * 8 + k], 2)
                    rows.append(table_ref[pl.ds(idx2, 2), :])
                lo = jnp.concatenate([r[0:1, :] for r in rows], axis=0)
                hi = jnp.concatenate([r[1:2, :] for r in rows], axis=0)
                out_ref[pl.ds(pl.multiple_of(base + g * 8, 8), 8), 0:128] = lo
                out_ref[pl.ds(pl.multiple_of(base + g * 8, 8), 8), 128:256] = hi
            return carry

        lax.fori_loop(0, _CHUNK // _UNROLL, body, 0)


def _gather_tiles(ids3, table2):
    n_tiles, _, tile = ids3.shape
    V2, _ = table2.shape
    return pl.pallas_call(
        _gather_kernel,
        out_shape=jax.ShapeDtypeStruct((n_tiles * tile, 256), jnp.float32),
        grid=(n_tiles,),
        in_specs=[
            pl.BlockSpec((1, 1, tile), lambda i: (i, 0, 0)),
            pl.BlockSpec((V2, 128), lambda i: (0, 0),
                         pipeline_mode=pl.Buffered(1)),
        ],
        out_specs=pl.BlockSpec((tile, 256), lambda i: (i, 0)),
        scratch_shapes=[
            pltpu.SMEM((tile,), jnp.int32),
            pltpu.SemaphoreType.DMA((tile // _CHUNK,)),
        ],
        compiler_params=pltpu.CompilerParams(
            dimension_semantics=("arbitrary",),
            vmem_limit_bytes=_VMEM_LIMIT,
        ),
    )(ids3, table2)


@jax.jit
def _emoji_gather(emojis, table):
    B, S = emojis.shape
    V, D = table.shape
    n = B * S
    p = D // 128

    ids = emojis.reshape(-1).astype(jnp.int32) * p
    tile = _TILE if n >= _TILE else pl.cdiv(n, _CHUNK) * _CHUNK
    n_pad = pl.cdiv(n, tile) * tile
    n_tiles = n_pad // tile
    if n_pad != n:
        ids = jnp.pad(ids, (0, n_pad - n))
    ids3 = ids.reshape(n_tiles, 1, tile)
    table2 = table.astype(jnp.float32).reshape(V * p, 128)

    ndev = len(jax.devices())
    if ndev > 1 and n_pad == n and n_tiles % ndev == 0 and B % ndev == 0:
        mesh = jax.make_mesh(
            (ndev,), ("c",),
            axis_types=(jax.sharding.AxisType.Explicit,),
        )
        ids3 = jax.reshard(ids3, jax.NamedSharding(mesh, P("c", None, None)))
        table2 = jax.reshard(table2, jax.NamedSharding(mesh, P()))

        def _local(i3, t2):
            return _gather_tiles(i3, t2).reshape(B // ndev, S, D)

        return jax.shard_map(
            _local, mesh=mesh, in_specs=(P("c"), P()), out_specs=P("c"),
            check_vma=False,
        )(ids3, table2)

    out = _gather_tiles(ids3, table2)
    if n_pad != n:
        out = out[:n]
    return out.reshape(B, S, D)


def kernel(emojis, table):
    return _emoji_gather(emojis, table)
